# baseline (speedup 1.0000x reference)
import functools

import jax
import jax.numpy as jnp
from jax import lax
from jax.experimental import pallas as pl
from jax.experimental.pallas import tpu as pltpu
from jax.experimental.pallas import tpu_sc as plsc

B = 16384
D = 32
NT = 4

_info = plsc.get_sparse_core_info()
NC = _info.num_cores
NS = _info.num_subcores
NW = NC * NS
BPW = B // NW
CHUNK = 128
NCH = BPW // CHUNK


@functools.partial(
    pl.kernel,
    mesh=plsc.VectorSubcoreMesh(core_axis_name="c", subcore_axis_name="s"),
    out_type=jax.ShapeDtypeStruct((NT * B, D), jnp.float32),
    compiler_params=pltpu.CompilerParams(use_tc_tiling_on_sc=False),
    scratch_types=[
        pltpu.VMEM((NT * NCH, CHUNK), jnp.int32),
        pltpu.VMEM((NT, BPW, D), jnp.float32),
        pltpu.SemaphoreType.DMA,
        pltpu.SemaphoreType.DMA,
    ],
)
def _emb_lookup(cat_i, sub_i, elem_i, evt_i,
                cat_t, sub_t, elem_t, evt_t,
                out_hbm, idx_v, rows_v, sem_i, sem_g):
    wid = lax.axis_index("s") * NC + lax.axis_index("c")
    base = wid * BPW
    idx_hbms = (cat_i, sub_i, elem_i, evt_i)
    tables = (cat_t, sub_t, elem_t, evt_t)

    idx_copies = []
    for t in range(NT):
        for j in range(NCH):
            idx_copies.append(pltpu.async_copy(
                idx_hbms[t].at[pl.ds(base + j * CHUNK, CHUNK)],
                idx_v.at[t * NCH + j], sem_i))
    for c in idx_copies:
        c.wait()

    gathers = []
    for t in range(NT):
        for j in range(NCH):
            gathers.append(pltpu.async_copy(
                tables[t].at[idx_v.at[t * NCH + j]],
                rows_v.at[t, pl.ds(j * CHUNK, CHUNK)], sem_g))
    for c in gathers:
        c.wait()

    writes = []
    for t in range(NT):
        writes.append(pltpu.async_copy(
            rows_v.at[t], out_hbm.at[pl.ds(t * B + base, BPW)], sem_i))
    for c in writes:
        c.wait()


def kernel(categories, sub_categories, elements, event_types,
           category_table, sub_category_table, element_table, event_type_table):
    return _emb_lookup(
        categories.astype(jnp.int32),
        sub_categories.astype(jnp.int32),
        elements.astype(jnp.int32),
        event_types.astype(jnp.int32),
        category_table, sub_category_table, element_table, event_type_table)


# speedup vs baseline: 1.9769x; 1.9769x over previous
import functools

import jax
import jax.numpy as jnp
from jax import lax
from jax.experimental import pallas as pl
from jax.experimental.pallas import tpu as pltpu
from jax.experimental.pallas import tpu_sc as plsc

B = 16384
D = 32
NT = 4

_info = plsc.get_sparse_core_info()
NC = _info.num_cores
NS = _info.num_subcores
NW = NC * NS
BPW = B // NW
NG = BPW // 16


def _iota16():
    return lax.iota(jnp.int32, 16)


def _kernel_decorator(interpret=False):
    return functools.partial(
        pl.kernel,
        mesh=plsc.VectorSubcoreMesh(core_axis_name="c", subcore_axis_name="s"),
        out_type=jax.ShapeDtypeStruct((D, NT * B), jnp.float32),
        compiler_params=pltpu.CompilerParams(needs_layout_passes=False),
        interpret=interpret,
        scratch_types=[
            pltpu.VMEM((BPW,), jnp.int32),
            pltpu.VMEM((16, 128), jnp.float32),
            pltpu.VMEM((16, 8, 128), jnp.float32),
            pltpu.VMEM((16, 4, 8, 128), jnp.float32),
            pltpu.SemaphoreType.DMA,
            pltpu.SemaphoreType.DMA,
        ],
    )


def _emb_body(cat_i, sub_i, elem_i, evt_i,
                cat_rm, sub_rm, elem_t3, evt_rm,
                out_hbm, idx_v, rows_v, blk_v, ring_v, sem_a, sem_b):
    wid = lax.axis_index("s") * NC + lax.axis_index("c")
    base = wid * BPW
    i16 = _iota16()
    tile_a = lax.shift_right_logical(i16, 3)
    tile_b = tile_a + 2
    sub_a = lax.rem(i16, 8)
    sub_b = sub_a

    def write_block_out(tbl_slot):
        copies = []
        for ta in range(4):
            for tb in range(4):
                copies.append(pltpu.async_copy(
                    blk_v.at[ta * 4 + tb],
                    out_hbm.at[pl.ds(ta * 8, 8),
                               pl.ds(tbl_slot * B + base + tb * 128, 128)],
                    sem_b))
        for cp in copies:
            cp.wait()

    def scatter_col(j, v0, v1):
        tb = lax.div(j, 128)
        jl = lax.rem(j, 128)
        plsc.store_scatter(
            blk_v, [tile_a * 4 + tb, sub_a, jnp.full((16,), jl, jnp.int32)], v0)
        plsc.store_scatter(
            blk_v, [tile_b * 4 + tb, sub_b, jnp.full((16,), jl, jnp.int32)], v1)

    for tbl, idx_hbm, slot in ((cat_rm, cat_i, 0), (sub_rm, sub_i, 1),
                               (evt_rm, evt_i, 3)):
        pltpu.async_copy(idx_hbm.at[pl.ds(base, BPW)], idx_v, sem_a).wait()

        def sr_group(g, carry, tbl=tbl):
            rv = plsc.load_gather(idx_v, [g * 16 + i16])
            sv = lax.shift_right_logical(rv, 2)
            pltpu.async_copy(tbl.at[sv], rows_v, sem_a).wait()
            zero16 = i16 * 0
            for k in range(16):
                r = rv[k]
                q = lax.rem(r, 4)
                off = q * 32
                v0 = plsc.load_gather(rows_v, [zero16 + k, off + i16])
                v1 = plsc.load_gather(rows_v, [zero16 + k, off + 16 + i16])
                scatter_col(g * 16 + k, v0, v1)
            return carry

        lax.fori_loop(0, NG, sr_group, 0)
        write_block_out(slot)

    pltpu.async_copy(elem_i.at[pl.ds(base, BPW)], idx_v, sem_a).wait()

    def el_group(g, carry):
        rv = plsc.load_gather(idx_v, [g * 16 + i16])
        fired = []
        for k in range(16):
            r = rv[k]
            col = pl.multiple_of(
                lax.shift_right_logical(r, 7) * 128, 128)
            fired.append(pltpu.async_copy(
                elem_t3.at[:, :, pl.ds(col, 128)], ring_v.at[k], sem_b))
        for k in range(16):
            fired[k].wait()
        zero16 = i16 * 0
        for k in range(16):
            r = rv[k]
            lane = lax.rem(r, 128)
            kf = zero16 + k
            lf = zero16 + lane
            v0 = plsc.load_gather(ring_v, [kf, tile_a, sub_a, lf])
            v1 = plsc.load_gather(ring_v, [kf, tile_b, sub_b, lf])
            scatter_col(g * 16 + k, v0, v1)
        return carry

    lax.fori_loop(0, NG, el_group, 0)
    write_block_out(2)


_emb_lookup = _kernel_decorator()(_emb_body)


def kernel(categories, sub_categories, elements, event_types,
           category_table, sub_category_table, element_table, event_type_table):
    out_t = _emb_lookup(
        categories.astype(jnp.int32),
        sub_categories.astype(jnp.int32),
        elements.astype(jnp.int32),
        event_types.astype(jnp.int32),
        category_table.reshape(250, 128),
        sub_category_table.reshape(25000, 128),
        element_table.T.reshape(4, 8, 1000000),
        event_type_table.reshape(250, 128))
    return out_t.T


# speedup vs baseline: 1.9914x; 1.0073x over previous
import functools

import jax
import jax.numpy as jnp
from jax import lax
from jax.experimental import pallas as pl
from jax.experimental.pallas import tpu as pltpu
from jax.experimental.pallas import tpu_sc as plsc

B = 16384
D = 32
NT = 4

_info = plsc.get_sparse_core_info()
NC = _info.num_cores
NS = _info.num_subcores
NW = NC * NS
BPW = B // NW
NG = BPW // 16


def _iota16():
    return lax.iota(jnp.int32, 16)


def _kernel_decorator(interpret=False):
    return functools.partial(
        pl.kernel,
        mesh=plsc.VectorSubcoreMesh(core_axis_name="c", subcore_axis_name="s"),
        out_type=jax.ShapeDtypeStruct((D, NT * B), jnp.float32),
        compiler_params=pltpu.CompilerParams(needs_layout_passes=False),
        interpret=interpret,
        scratch_types=[
            pltpu.VMEM((BPW,), jnp.int32),
            pltpu.VMEM((16, 128), jnp.float32),
            pltpu.VMEM((16, 8, 128), jnp.float32),
            pltpu.VMEM((2, 8, 4, 8, 128), jnp.float32),
            pltpu.SemaphoreType.DMA,
            pltpu.SemaphoreType.DMA,
            pltpu.SemaphoreType.DMA,
        ],
    )


def _emb_body(cat_i, sub_i, elem_i, evt_i,
                cat_rm, sub_rm, elem_t3, evt_rm,
                out_hbm, idx_v, rows_v, blk_v, ring_v, sem_a, sem_b, sem_c):
    wid = lax.axis_index("s") * NC + lax.axis_index("c")
    base = wid * BPW
    i16 = _iota16()
    tile_a = lax.shift_right_logical(i16, 3)
    tile_b = tile_a + 2
    sub_a = lax.rem(i16, 8)
    sub_b = sub_a

    def write_block_out(tbl_slot):
        copies = []
        for ta in range(4):
            for tb in range(4):
                copies.append(pltpu.async_copy(
                    blk_v.at[ta * 4 + tb],
                    out_hbm.at[pl.ds(ta * 8, 8),
                               pl.ds(tbl_slot * B + base + tb * 128, 128)],
                    sem_b))
        for cp in copies:
            cp.wait()

    def scatter_col(j, v0, v1):
        tb = lax.div(j, 128)
        jl = lax.rem(j, 128)
        plsc.store_scatter(
            blk_v, [tile_a * 4 + tb, sub_a, jnp.full((16,), jl, jnp.int32)], v0)
        plsc.store_scatter(
            blk_v, [tile_b * 4 + tb, sub_b, jnp.full((16,), jl, jnp.int32)], v1)

    for tbl, idx_hbm, slot in ((cat_rm, cat_i, 0), (sub_rm, sub_i, 1),
                               (evt_rm, evt_i, 3)):
        pltpu.async_copy(idx_hbm.at[pl.ds(base, BPW)], idx_v, sem_a).wait()

        def sr_group(g, carry, tbl=tbl):
            rv = plsc.load_gather(idx_v, [g * 16 + i16])
            sv = lax.shift_right_logical(rv, 2)
            pltpu.async_copy(tbl.at[sv], rows_v, sem_a).wait()
            zero16 = i16 * 0
            for k in range(16):
                r = rv[k]
                q = lax.rem(r, 4)
                off = q * 32
                v0 = plsc.load_gather(rows_v, [zero16 + k, off + i16])
                v1 = plsc.load_gather(rows_v, [zero16 + k, off + 16 + i16])
                scatter_col(g * 16 + k, v0, v1)
            return carry

        lax.fori_loop(0, NG, sr_group, 0)
        write_block_out(slot)

    pltpu.async_copy(elem_i.at[pl.ds(base, BPW)], idx_v, sem_a).wait()
    zero16 = i16 * 0
    NH = BPW // 8

    def el_rv(h):
        return plsc.load_gather(idx_v, [lax.rem(h * 8 + i16, BPW)])

    def el_issue(h, slot, sem):
        rv = el_rv(h)
        for k in range(8):
            col = pl.multiple_of(
                lax.shift_right_logical(rv[k], 7) * 128, 128)
            pltpu.async_copy(
                elem_t3.at[:, :, pl.ds(col, 128)], ring_v.at[slot, k], sem)

    def el_drain(slot, sem):
        for k in range(8):
            pltpu.make_async_copy(
                elem_t3.at[:, :, pl.ds(0, 128)], ring_v.at[slot, k], sem
            ).wait()

    def el_extract(h, slot):
        rv = el_rv(h)
        for k in range(8):
            lane = lax.rem(rv[k], 128)
            kf = zero16 + k
            sf = zero16 + slot
            lf = zero16 + lane
            v0 = plsc.load_gather(ring_v, [sf, kf, tile_a, sub_a, lf])
            v1 = plsc.load_gather(ring_v, [sf, kf, tile_b, sub_b, lf])
            scatter_col(h * 8 + k, v0, v1)

    el_issue(0, 0, sem_b)

    def el_pair(p, carry):
        h0 = p * 2
        el_issue(h0 + 1, 1, sem_c)
        el_drain(0, sem_b)
        el_extract(h0, 0)
        el_issue(lax.min(h0 + 2, NH - 1), 0, sem_b)
        el_drain(1, sem_c)
        el_extract(h0 + 1, 1)
        return carry

    lax.fori_loop(0, NH // 2, el_pair, 0)
    el_drain(0, sem_b)
    write_block_out(2)


_emb_lookup = _kernel_decorator()(_emb_body)


def kernel(categories, sub_categories, elements, event_types,
           category_table, sub_category_table, element_table, event_type_table):
    out_t = _emb_lookup(
        categories.astype(jnp.int32),
        sub_categories.astype(jnp.int32),
        elements.astype(jnp.int32),
        event_types.astype(jnp.int32),
        category_table.reshape(250, 128),
        sub_category_table.reshape(25000, 128),
        element_table.T.reshape(4, 8, 1000000),
        event_type_table.reshape(250, 128))
    return out_t.T
